# Initial kernel scaffold; baseline (speedup 1.0000x reference)
import jax
import jax.numpy as jnp
from jax.experimental import pallas as pl


def kernel(x, edge_index, edge_attr, batch, params):
    raise NotImplementedError("write your pallas kernel here")


# speedup vs baseline: 5.9910x; 5.9910x over previous
import functools

import jax
import jax.numpy as jnp
from jax import lax
from jax.experimental import pallas as pl
from jax.experimental.pallas import tpu as pltpu
from jax.experimental.pallas import tpu_sc as plsc

_N = 10000
_E = 320000
_NE = 16
_NOUT1 = 32
_NOUT2 = 16
_NG = 32

_NC = 2
_NS = 16
_NW = _NC * _NS
_EPW = _E // _NW
_CH = 80
_NCHUNK = _EPW // _CH
_NP = 10240
_NSH = _NP // _NS


def _sc_mesh():
    return plsc.VectorSubcoreMesh(
        core_axis_name="c", subcore_axis_name="s",
        num_cores=_NC, num_subcores=_NS)


@functools.partial(jax.jit, static_argnames=("d",))
def _sc_gather(table, idx, d):
    @functools.partial(
        pl.kernel,
        out_type=jax.ShapeDtypeStruct((_E, d), jnp.float32),
        mesh=_sc_mesh(),
        scratch_types=[
            pltpu.VMEM((_CH,), jnp.int32),
            pltpu.VMEM((_CH, d), jnp.float32),
            pltpu.SemaphoreType.DMA,
        ],
    )
    def k(table_hbm, idx_hbm, out_hbm, idx_v, rows_v, sem):
        wid = lax.axis_index("s") * _NC + lax.axis_index("c")
        base = wid * _EPW

        def body(c, carry):
            off = base + c * _CH
            pltpu.sync_copy(idx_hbm.at[pl.ds(off, _CH)], idx_v)
            pltpu.async_copy(table_hbm.at[idx_v], rows_v, sem).wait()
            pltpu.sync_copy(rows_v, out_hbm.at[pl.ds(off, _CH)])
            return carry

        lax.fori_loop(0, _NCHUNK, body, 0)

    return k(table, idx)


_HALF = _NP // 2
_ACC = _HALF + 8
_RSH = _HALF // _NS
_EPT = _E // _NS
_NCH2 = _EPT // _CH


@jax.jit
def _sc_scatter(m, dst, zeros_n):
    @functools.partial(
        pl.kernel,
        out_type=jax.ShapeDtypeStruct((_NP, 128), jnp.float32),
        mesh=_sc_mesh(),
        scratch_types=[
            pltpu.VMEM((_CH,), jnp.int32),
            pltpu.VMEM((_CH,), jnp.int32),
            pltpu.VMEM((_CH, 128), jnp.float32),
            pltpu.VMEM((_RSH, 128), jnp.float32),
            pltpu.VMEM_SHARED((_ACC, 128), jnp.float32),
            pltpu.SemaphoreType.DMA,
        ],
    )
    def k(m_hbm, dst_hbm, zero_hbm, out_hbm, idx_v, idx2_v, rows_v, shard_v,
          acc_sh, sem):
        c = lax.axis_index("c")
        s = lax.axis_index("s")
        base = s * _EPT
        sh0 = s * _RSH
        lo = c * _HALF

        pltpu.sync_copy(zero_hbm.at[pl.ds(sh0, _RSH)], shard_v)
        pltpu.sync_copy(shard_v, acc_sh.at[pl.ds(sh0, _RSH)])
        plsc.subcore_barrier()

        def body(ci, carry):
            off = base + ci * _CH
            pltpu.sync_copy(dst_hbm.at[pl.ds(off, _CH)], idx_v)
            pltpu.sync_copy(m_hbm.at[pl.ds(off, _CH)], rows_v)
            for j in range(_CH // 16):
                v = idx_v[pl.ds(j * 16, 16)] - lo
                ok = (v >= 0) & (v < _HALF)
                idx2_v[pl.ds(j * 16, 16)] = jnp.where(ok, v, _HALF)
            pltpu.sync_copy(rows_v, acc_sh.at[idx2_v], add=True)
            return carry

        lax.fori_loop(0, _NCH2, body, 0)
        plsc.subcore_barrier()

        pltpu.sync_copy(acc_sh.at[pl.ds(sh0, _RSH)], shard_v)
        pltpu.sync_copy(shard_v, out_hbm.at[pl.ds(lo + sh0, _RSH)])

    return k(m, dst, zeros_n)


def _scatter_dispatch(m, dst, zeros_n):
    return _sc_scatter(m, dst, zeros_n)


_TE = 2000


def _edge_body(attr_ref, xj_ref, f1_ref, f2_ref, f3_ref, f4_ref, w_ref, m_ref):
    attr = attr_ref[...]
    t1 = jnp.maximum(attr @ f1_ref[...], 0.0)
    t2 = jnp.tanh(attr @ f2_ref[...]) * jnp.tanh(attr @ f3_ref[...])
    tmp = jnp.concatenate([t1, t2], axis=1)
    ea = jnp.maximum(tmp @ f4_ref[...], 0.0)
    z = jnp.dot(xj_ref[...], w_ref[...], preferred_element_type=jnp.float32)
    acc = ea[:, 0:1] * z[:, 0:_NOUT1]
    for i in range(1, _NE):
        acc = acc + ea[:, i:i + 1] * z[:, i * _NOUT1:(i + 1) * _NOUT1]
    mpad = jnp.zeros((acc.shape[0], 128 - _NOUT1), jnp.float32)
    m_ref[...] = jnp.concatenate([acc, mpad], axis=1)


@functools.partial(jax.jit, static_argnames=("ninp",))
def _edge_messages(edge_attr, xj, f1, f2, f3, f4, w_all, ninp):
    grid = _E // _TE
    return pl.pallas_call(
        _edge_body,
        grid=(grid,),
        in_specs=[
            pl.BlockSpec((_TE, _NE), lambda i: (i, 0)),
            pl.BlockSpec((_TE, ninp), lambda i: (i, 0)),
            pl.BlockSpec((_NE, 2 * _NE), lambda i: (0, 0)),
            pl.BlockSpec((_NE, 2 * _NE), lambda i: (0, 0)),
            pl.BlockSpec((_NE, 2 * _NE), lambda i: (0, 0)),
            pl.BlockSpec((4 * _NE, _NE), lambda i: (0, 0)),
            pl.BlockSpec((ninp, _NE * _NOUT1), lambda i: (0, 0)),
        ],
        out_specs=pl.BlockSpec((_TE, 128), lambda i: (i, 0)),
        out_shape=jax.ShapeDtypeStruct((_E, 128), jnp.float32),
    )(edge_attr, xj, f1, f2, f3, f4, w_all)


_TN = 2000


def _node_body(agg2_ref, x_ref, cb_ref, f11_ref, b11_ref, f12_ref, b12_ref, h_ref):
    agg = agg2_ref[:, 0:_NOUT1]
    out1 = jnp.maximum(agg + cb_ref[...], 0.0)
    xt = x_ref[...]
    a = jnp.tanh(jnp.dot(xt, f11_ref[...], preferred_element_type=jnp.float32) + b11_ref[...])
    b = jnp.tanh(jnp.dot(xt, f12_ref[...], preferred_element_type=jnp.float32) + b12_ref[...])
    pad = jnp.zeros((out1.shape[0], 128 - _NOUT1 - _NOUT2), jnp.float32)
    h_ref[...] = jnp.concatenate([out1, a * b, pad], axis=1)


@functools.partial(jax.jit, static_argnames=("ninp",))
def _node_update(agg2, x, cb, f11, b11, f12, b12, ninp):
    grid = _N // _TN
    return pl.pallas_call(
        _node_body,
        grid=(grid,),
        in_specs=[
            pl.BlockSpec((_TN, 128), lambda i: (i, 0)),
            pl.BlockSpec((_TN, ninp), lambda i: (i, 0)),
            pl.BlockSpec((1, _NOUT1), lambda i: (0, 0)),
            pl.BlockSpec((ninp, _NOUT2), lambda i: (0, 0)),
            pl.BlockSpec((1, _NOUT2), lambda i: (0, 0)),
            pl.BlockSpec((ninp, _NOUT2), lambda i: (0, 0)),
            pl.BlockSpec((1, _NOUT2), lambda i: (0, 0)),
        ],
        out_specs=pl.BlockSpec((_TN, 128), lambda i: (i, 0)),
        out_shape=jax.ShapeDtypeStruct((_N, 128), jnp.float32),
    )(agg2, x, cb, f11, b11, f12, b12)


def _head_body(h_ref, batch_ref, w1_ref, b1_ref, w2_ref, b2_ref, out_ref,
               sums_ref, cnts_ref):
    i = pl.program_id(0)

    @pl.when(i == 0)
    def _():
        sums_ref[...] = jnp.zeros_like(sums_ref)
        cnts_ref[...] = jnp.zeros_like(cnts_ref)

    onehot_t = (lax.broadcasted_iota(jnp.int32, (_NG, _TN), 0)
                == batch_ref[0]).astype(jnp.float32)
    sums_ref[...] += jnp.dot(onehot_t, h_ref[...], preferred_element_type=jnp.float32)
    cnts_ref[...] += jnp.sum(onehot_t, axis=1, keepdims=True)

    @pl.when(i == pl.num_programs(0) - 1)
    def _():
        g = sums_ref[...] / jnp.maximum(cnts_ref[...], 1.0)
        a = jnp.maximum(jnp.dot(g, w1_ref[...], preferred_element_type=jnp.float32)
                        + b1_ref[...], 0.0)
        out_ref[...] = jnp.dot(a, w2_ref[...], preferred_element_type=jnp.float32) + b2_ref[...]


@jax.jit
def _head(h, batch_row, w1, b1, w2, b2):
    grid = _N // _TN
    nin = 128
    return pl.pallas_call(
        _head_body,
        grid=(grid,),
        in_specs=[
            pl.BlockSpec((_TN, nin), lambda i: (i, 0)),
            pl.BlockSpec((1, 1, _TN), lambda i: (i, 0, 0)),
            pl.BlockSpec((nin, 10), lambda i: (0, 0)),
            pl.BlockSpec((1, 10), lambda i: (0, 0)),
            pl.BlockSpec((10, 1), lambda i: (0, 0)),
            pl.BlockSpec((1, 1), lambda i: (0, 0)),
        ],
        out_specs=pl.BlockSpec((_NG, 1), lambda i: (0, 0)),
        out_shape=jax.ShapeDtypeStruct((_NG, 1), jnp.float32),
        scratch_shapes=[
            pltpu.VMEM((_NG, 128), jnp.float32),
            pltpu.VMEM((_NG, 1), jnp.float32),
        ],
    )(h, batch_row, w1, b1, w2, b2)


def kernel(x, edge_index, edge_attr, batch, params):
    src = edge_index[0]
    dst = edge_index[1]
    zeros_n = jnp.zeros((_NP, 128), jnp.float32)
    batch_row = batch.reshape(_N // _TN, 1, _TN)

    def pad_rows(w):
        return jnp.pad(w, ((0, 128 - w.shape[0]), (0, 0)))

    h = x
    for lp in (params['l1'], params['l2'], params['l3']):
        ninp = lp['conv_w'].shape[1]
        w_all = pad_rows(jnp.transpose(lp['conv_w'], (1, 0, 2)).reshape(ninp, _NE * _NOUT1))
        xj = _sc_gather(h, src, 128)
        m = _edge_messages(edge_attr, xj, lp['fc1_1'], lp['fc1_2'], lp['fc1_3'],
                           lp['fc1_4'], w_all, 128)
        agg2 = _scatter_dispatch(m, dst, zeros_n)
        h = _node_update(agg2, h, lp['conv_b'].reshape(1, _NOUT1),
                         pad_rows(lp['fc11_w']), lp['fc11_b'].reshape(1, _NOUT2),
                         pad_rows(lp['fc12_w']), lp['fc12_b'].reshape(1, _NOUT2), 128)

    return _head(h, batch_row, pad_rows(params['fc1_w']), params['fc1_b'].reshape(1, 10),
                 params['fc2_w'], params['fc2_b'].reshape(1, 1))


# speedup vs baseline: 10.6741x; 1.7817x over previous
import functools

import jax
import jax.numpy as jnp
from jax import lax
from jax.experimental import pallas as pl
from jax.experimental.pallas import tpu as pltpu
from jax.experimental.pallas import tpu_sc as plsc

_N = 10000
_E = 320000
_NE = 16
_NOUT1 = 32
_NOUT2 = 16
_NG = 32

_NC = 2
_NS = 16
_NW = _NC * _NS
_EPW = _E // _NW
_CH = 80
_NCHUNK = _EPW // _CH
_NP = 10240
_NSH = _NP // _NS


def _sc_mesh():
    return plsc.VectorSubcoreMesh(
        core_axis_name="c", subcore_axis_name="s",
        num_cores=_NC, num_subcores=_NS)


@functools.partial(jax.jit, static_argnames=("d",))
def _sc_gather(table, idx, d):
    @functools.partial(
        pl.kernel,
        out_type=jax.ShapeDtypeStruct((_E, d), jnp.float32),
        mesh=_sc_mesh(),
        scratch_types=[
            pltpu.VMEM((_CH,), jnp.int32),
            pltpu.VMEM((_CH, d), jnp.float32),
            pltpu.SemaphoreType.DMA,
        ],
    )
    def k(table_hbm, idx_hbm, out_hbm, idx_v, rows_v, sem):
        wid = lax.axis_index("s") * _NC + lax.axis_index("c")
        base = wid * _EPW

        def body(c, carry):
            off = base + c * _CH
            pltpu.sync_copy(idx_hbm.at[pl.ds(off, _CH)], idx_v)
            pltpu.async_copy(table_hbm.at[idx_v], rows_v, sem).wait()
            pltpu.sync_copy(rows_v, out_hbm.at[pl.ds(off, _CH)])
            return carry

        lax.fori_loop(0, _NCHUNK, body, 0)

    return k(table, idx)


_HALF = _NP // 2
_ACC = _HALF + 8
_RSH = _HALF // _NS
_EPT = _E // _NS
_NCH2 = _EPT // _CH


@jax.jit
def _sc_scatter(m, dst, zeros_n):
    @functools.partial(
        pl.kernel,
        out_type=jax.ShapeDtypeStruct((_NP, 128), jnp.float32),
        mesh=_sc_mesh(),
        scratch_types=[
            pltpu.VMEM((_CH,), jnp.int32),
            pltpu.VMEM((_CH,), jnp.int32),
            pltpu.VMEM((_CH, 128), jnp.float32),
            pltpu.VMEM((_RSH, 128), jnp.float32),
            pltpu.VMEM_SHARED((_ACC, 128), jnp.float32),
            pltpu.SemaphoreType.DMA,
        ],
    )
    def k(m_hbm, dst_hbm, zero_hbm, out_hbm, idx_v, idx2_v, rows_v, shard_v,
          acc_sh, sem):
        c = lax.axis_index("c")
        s = lax.axis_index("s")
        base = s * _EPT
        sh0 = s * _RSH
        lo = c * _HALF

        pltpu.sync_copy(zero_hbm.at[pl.ds(sh0, _RSH)], shard_v)
        pltpu.sync_copy(shard_v, acc_sh.at[pl.ds(sh0, _RSH)])
        plsc.subcore_barrier()

        def body(ci, carry):
            off = base + ci * _CH
            pltpu.sync_copy(dst_hbm.at[pl.ds(off, _CH)], idx_v)
            pltpu.sync_copy(m_hbm.at[pl.ds(off, _CH)], rows_v)
            for j in range(_CH // 16):
                v = idx_v[pl.ds(j * 16, 16)] - lo
                ok = (v >= 0) & (v < _HALF)
                idx2_v[pl.ds(j * 16, 16)] = jnp.where(ok, v, _HALF)
            pltpu.sync_copy(rows_v, acc_sh.at[idx2_v], add=True)
            return carry

        lax.fori_loop(0, _NCH2, body, 0)
        plsc.subcore_barrier()

        pltpu.sync_copy(acc_sh.at[pl.ds(sh0, _RSH)], shard_v)
        pltpu.sync_copy(shard_v, out_hbm.at[pl.ds(lo + sh0, _RSH)])

    return k(m, dst, zeros_n)


def _scatter_dispatch(m, dst, zeros_n):
    return _sc_scatter(m, dst, zeros_n)


_TE = 2000


def _edge_body(attr_ref, xj_ref, f1_ref, f2_ref, f3_ref, f4_ref, w_ref,
               r_ref, s_ref, m_ref):
    attr = attr_ref[...]
    t1 = jnp.maximum(attr @ f1_ref[...], 0.0)
    t2 = jnp.tanh(attr @ f2_ref[...]) * jnp.tanh(attr @ f3_ref[...])
    tmp = jnp.concatenate([t1, t2], axis=1)
    ea = jnp.maximum(tmp @ f4_ref[...], 0.0)
    z = jnp.dot(xj_ref[...], w_ref[...], preferred_element_type=jnp.float32)
    ea_rep = jnp.dot(ea, r_ref[...], preferred_element_type=jnp.float32)
    acc = jnp.dot(z * ea_rep, s_ref[...], preferred_element_type=jnp.float32)
    mpad = jnp.zeros((acc.shape[0], 128 - _NOUT1), jnp.float32)
    m_ref[...] = jnp.concatenate([acc, mpad], axis=1)


@functools.partial(jax.jit, static_argnames=("ninp",))
def _edge_messages(edge_attr, xj, f1, f2, f3, f4, w_all, rep_mat, sum_mat, ninp):
    grid = _E // _TE
    return pl.pallas_call(
        _edge_body,
        grid=(grid,),
        in_specs=[
            pl.BlockSpec((_TE, _NE), lambda i: (i, 0)),
            pl.BlockSpec((_TE, ninp), lambda i: (i, 0)),
            pl.BlockSpec((_NE, 2 * _NE), lambda i: (0, 0)),
            pl.BlockSpec((_NE, 2 * _NE), lambda i: (0, 0)),
            pl.BlockSpec((_NE, 2 * _NE), lambda i: (0, 0)),
            pl.BlockSpec((4 * _NE, _NE), lambda i: (0, 0)),
            pl.BlockSpec((ninp, _NE * _NOUT1), lambda i: (0, 0)),
            pl.BlockSpec((_NE, _NE * _NOUT1), lambda i: (0, 0)),
            pl.BlockSpec((_NE * _NOUT1, _NOUT1), lambda i: (0, 0)),
        ],
        out_specs=pl.BlockSpec((_TE, 128), lambda i: (i, 0)),
        out_shape=jax.ShapeDtypeStruct((_E, 128), jnp.float32),
    )(edge_attr, xj, f1, f2, f3, f4, w_all, rep_mat, sum_mat)


_TN = 2000


def _node_body(agg2_ref, x_ref, cb_ref, f11_ref, b11_ref, f12_ref, b12_ref, h_ref):
    agg = agg2_ref[:, 0:_NOUT1]
    out1 = jnp.maximum(agg + cb_ref[...], 0.0)
    xt = x_ref[...]
    a = jnp.tanh(jnp.dot(xt, f11_ref[...], preferred_element_type=jnp.float32) + b11_ref[...])
    b = jnp.tanh(jnp.dot(xt, f12_ref[...], preferred_element_type=jnp.float32) + b12_ref[...])
    pad = jnp.zeros((out1.shape[0], 128 - _NOUT1 - _NOUT2), jnp.float32)
    h_ref[...] = jnp.concatenate([out1, a * b, pad], axis=1)


@functools.partial(jax.jit, static_argnames=("ninp",))
def _node_update(agg2, x, cb, f11, b11, f12, b12, ninp):
    grid = _N // _TN
    return pl.pallas_call(
        _node_body,
        grid=(grid,),
        in_specs=[
            pl.BlockSpec((_TN, 128), lambda i: (i, 0)),
            pl.BlockSpec((_TN, ninp), lambda i: (i, 0)),
            pl.BlockSpec((1, _NOUT1), lambda i: (0, 0)),
            pl.BlockSpec((ninp, _NOUT2), lambda i: (0, 0)),
            pl.BlockSpec((1, _NOUT2), lambda i: (0, 0)),
            pl.BlockSpec((ninp, _NOUT2), lambda i: (0, 0)),
            pl.BlockSpec((1, _NOUT2), lambda i: (0, 0)),
        ],
        out_specs=pl.BlockSpec((_TN, 128), lambda i: (i, 0)),
        out_shape=jax.ShapeDtypeStruct((_N, 128), jnp.float32),
    )(agg2, x, cb, f11, b11, f12, b12)


def _head_body(h_ref, batch_ref, w1_ref, b1_ref, w2_ref, b2_ref, out_ref,
               sums_ref, cnts_ref):
    i = pl.program_id(0)

    @pl.when(i == 0)
    def _():
        sums_ref[...] = jnp.zeros_like(sums_ref)
        cnts_ref[...] = jnp.zeros_like(cnts_ref)

    onehot_t = (lax.broadcasted_iota(jnp.int32, (_NG, _TN), 0)
                == batch_ref[0]).astype(jnp.float32)
    sums_ref[...] += jnp.dot(onehot_t, h_ref[...], preferred_element_type=jnp.float32)
    cnts_ref[...] += jnp.sum(onehot_t, axis=1, keepdims=True)

    @pl.when(i == pl.num_programs(0) - 1)
    def _():
        g = sums_ref[...] / jnp.maximum(cnts_ref[...], 1.0)
        a = jnp.maximum(jnp.dot(g, w1_ref[...], preferred_element_type=jnp.float32)
                        + b1_ref[...], 0.0)
        out_ref[...] = jnp.dot(a, w2_ref[...], preferred_element_type=jnp.float32) + b2_ref[...]


@jax.jit
def _head(h, batch_row, w1, b1, w2, b2):
    grid = _N // _TN
    nin = 128
    return pl.pallas_call(
        _head_body,
        grid=(grid,),
        in_specs=[
            pl.BlockSpec((_TN, nin), lambda i: (i, 0)),
            pl.BlockSpec((1, 1, _TN), lambda i: (i, 0, 0)),
            pl.BlockSpec((nin, 10), lambda i: (0, 0)),
            pl.BlockSpec((1, 10), lambda i: (0, 0)),
            pl.BlockSpec((10, 1), lambda i: (0, 0)),
            pl.BlockSpec((1, 1), lambda i: (0, 0)),
        ],
        out_specs=pl.BlockSpec((_NG, 1), lambda i: (0, 0)),
        out_shape=jax.ShapeDtypeStruct((_NG, 1), jnp.float32),
        scratch_shapes=[
            pltpu.VMEM((_NG, 128), jnp.float32),
            pltpu.VMEM((_NG, 1), jnp.float32),
        ],
    )(h, batch_row, w1, b1, w2, b2)


def kernel(x, edge_index, edge_attr, batch, params):
    src = edge_index[0]
    dst = edge_index[1]
    zeros_n = jnp.zeros((_NP, 128), jnp.float32)
    batch_row = batch.reshape(_N // _TN, 1, _TN)

    ii = jnp.arange(_NE * _NOUT1, dtype=jnp.int32)
    rep_mat = (ii[None, :] // _NOUT1 == jnp.arange(_NE, dtype=jnp.int32)[:, None]
               ).astype(jnp.float32)
    sum_mat = (ii[:, None] % _NOUT1 == jnp.arange(_NOUT1, dtype=jnp.int32)[None, :]
               ).astype(jnp.float32)

    def pad_rows(w):
        return jnp.pad(w, ((0, 128 - w.shape[0]), (0, 0)))

    h = x
    for lp in (params['l1'], params['l2'], params['l3']):
        ninp = lp['conv_w'].shape[1]
        w_all = pad_rows(jnp.transpose(lp['conv_w'], (1, 0, 2)).reshape(ninp, _NE * _NOUT1))
        xj = _sc_gather(h, src, 128)
        m = _edge_messages(edge_attr, xj, lp['fc1_1'], lp['fc1_2'], lp['fc1_3'],
                           lp['fc1_4'], w_all, rep_mat, sum_mat, 128)
        agg2 = _scatter_dispatch(m, dst, zeros_n)
        h = _node_update(agg2, h, lp['conv_b'].reshape(1, _NOUT1),
                         pad_rows(lp['fc11_w']), lp['fc11_b'].reshape(1, _NOUT2),
                         pad_rows(lp['fc12_w']), lp['fc12_b'].reshape(1, _NOUT2), 128)

    return _head(h, batch_row, pad_rows(params['fc1_w']), params['fc1_b'].reshape(1, 10),
                 params['fc2_w'], params['fc2_b'].reshape(1, 1))


# speedup vs baseline: 14.6218x; 1.3698x over previous
import functools

import jax
import jax.numpy as jnp
from jax import lax
from jax.experimental import pallas as pl
from jax.experimental.pallas import tpu as pltpu
from jax.experimental.pallas import tpu_sc as plsc

_N = 10000
_E = 320000
_NE = 16
_NOUT1 = 32
_NOUT2 = 16
_NG = 32

_NC = 2
_NS = 16
_NW = _NC * _NS
_EPW = _E // _NW
_CH = 80
_NCHUNK = _EPW // _CH
_NP = 10240
_NSH = _NP // _NS


def _sc_mesh():
    return plsc.VectorSubcoreMesh(
        core_axis_name="c", subcore_axis_name="s",
        num_cores=_NC, num_subcores=_NS)


_NCHG = _EPW // _CH
_NOUTER_G = (_NCHG - 1) // 2


@functools.partial(jax.jit, static_argnames=("d",))
def _sc_gather(table, idx, d):
    @functools.partial(
        pl.kernel,
        out_type=jax.ShapeDtypeStruct((_E, d), jnp.float32),
        mesh=_sc_mesh(),
        scratch_types=[
            pltpu.VMEM((2, _CH), jnp.int32),
            pltpu.VMEM((2, _CH, d), jnp.float32),
            pltpu.SemaphoreType.DMA,
            pltpu.SemaphoreType.DMA,
            pltpu.SemaphoreType.DMA,
            pltpu.SemaphoreType.DMA,
            pltpu.SemaphoreType.DMA,
        ],
    )
    def k(table_hbm, idx_hbm, out_hbm, idx_v, rows_v, si0, si1, sg, ss0, ss1):
        wid = lax.axis_index("s") * _NC + lax.axis_index("c")
        base = wid * _EPW
        sidx = (si0, si1)
        sstore = (ss0, ss1)

        for b in (0, 1):
            pltpu.async_copy(idx_hbm.at[pl.ds(base + b * _CH, _CH)],
                             idx_v.at[b], sidx[b])

        def outer(t, carry):
            for b in (0, 1):
                ci = 2 * t + b
                off = base + ci * _CH
                pltpu.make_async_copy(idx_hbm.at[pl.ds(off, _CH)],
                                      idx_v.at[b], sidx[b]).wait()

                @pl.when(t > 0)
                def _():
                    pltpu.make_async_copy(rows_v.at[b],
                                          out_hbm.at[pl.ds(off, _CH)],
                                          sstore[b]).wait()

                pltpu.async_copy(table_hbm.at[idx_v.at[b]], rows_v.at[b], sg).wait()
                pltpu.async_copy(rows_v.at[b], out_hbm.at[pl.ds(off, _CH)],
                                 sstore[b])

                @pl.when(ci + 2 < _NCHG)
                def _():
                    pltpu.async_copy(idx_hbm.at[pl.ds(off + 2 * _CH, _CH)],
                                     idx_v.at[b], sidx[b])
            return carry

        lax.fori_loop(0, _NOUTER_G, outer, 0)

        off = base + (_NCHG - 1) * _CH
        pltpu.make_async_copy(idx_hbm.at[pl.ds(off, _CH)], idx_v.at[0], si0).wait()
        pltpu.make_async_copy(rows_v.at[0], out_hbm.at[pl.ds(off, _CH)], ss0).wait()
        pltpu.async_copy(table_hbm.at[idx_v.at[0]], rows_v.at[0], sg).wait()
        pltpu.sync_copy(rows_v.at[0], out_hbm.at[pl.ds(off, _CH)])
        pltpu.make_async_copy(rows_v.at[1], out_hbm.at[pl.ds(off, _CH)], ss1).wait()

    return k(table, idx)


_HALF = _NP // 2
_ACC = _HALF + 8
_RSH = _HALF // _NS
_EPT = _E // _NS
_NCH2 = _EPT // _CH


@jax.jit
def _sc_scatter(m, dst, zeros_n):
    @functools.partial(
        pl.kernel,
        out_type=jax.ShapeDtypeStruct((_NP, 128), jnp.float32),
        mesh=_sc_mesh(),
        scratch_types=[
            pltpu.VMEM((2, _CH), jnp.int32),
            pltpu.VMEM((2, _CH), jnp.int32),
            pltpu.VMEM((2, _CH, 128), jnp.float32),
            pltpu.VMEM((_RSH, 128), jnp.float32),
            pltpu.VMEM_SHARED((_ACC, 128), jnp.float32),
            pltpu.SemaphoreType.DMA,
            pltpu.SemaphoreType.DMA,
            pltpu.SemaphoreType.DMA,
            pltpu.SemaphoreType.DMA,
        ],
    )
    def k(m_hbm, dst_hbm, zero_hbm, out_hbm, idx_v, idx2_v, rows_v, shard_v,
          acc_sh, si0, si1, sr0, sr1):
        c = lax.axis_index("c")
        s = lax.axis_index("s")
        base = s * _EPT
        sh0 = s * _RSH
        lo = c * _HALF
        sidx = (si0, si1)
        srow = (sr0, sr1)

        pltpu.sync_copy(zero_hbm.at[pl.ds(sh0, _RSH)], shard_v)
        pltpu.sync_copy(shard_v, acc_sh.at[pl.ds(sh0, _RSH)])
        plsc.subcore_barrier()

        for b in (0, 1):
            pltpu.async_copy(dst_hbm.at[pl.ds(base + b * _CH, _CH)],
                             idx_v.at[b], sidx[b])
            pltpu.async_copy(m_hbm.at[pl.ds(base + b * _CH, _CH)],
                             rows_v.at[b], srow[b])

        def outer(t, carry):
            for b in (0, 1):
                ci = 2 * t + b
                off = base + ci * _CH
                pltpu.make_async_copy(dst_hbm.at[pl.ds(off, _CH)],
                                      idx_v.at[b], sidx[b]).wait()
                pltpu.make_async_copy(m_hbm.at[pl.ds(off, _CH)],
                                      rows_v.at[b], srow[b]).wait()
                for j in range(_CH // 16):
                    v = idx_v[b, pl.ds(j * 16, 16)] - lo
                    ok = (v >= 0) & (v < _HALF)
                    idx2_v[b, pl.ds(j * 16, 16)] = jnp.where(ok, v, _HALF)
                pltpu.sync_copy(rows_v.at[b], acc_sh.at[idx2_v.at[b]], add=True)

                @pl.when(ci + 2 < _NCH2)
                def _():
                    pltpu.async_copy(dst_hbm.at[pl.ds(off + 2 * _CH, _CH)],
                                     idx_v.at[b], sidx[b])
                    pltpu.async_copy(m_hbm.at[pl.ds(off + 2 * _CH, _CH)],
                                     rows_v.at[b], srow[b])
            return carry

        lax.fori_loop(0, _NCH2 // 2, outer, 0)
        plsc.subcore_barrier()

        pltpu.sync_copy(acc_sh.at[pl.ds(sh0, _RSH)], shard_v)
        pltpu.sync_copy(shard_v, out_hbm.at[pl.ds(lo + sh0, _RSH)])

    return k(m, dst, zeros_n)


def _scatter_dispatch(m, dst, zeros_n):
    return _sc_scatter(m, dst, zeros_n)


_TE = 2000


def _edge_body(attr_ref, xj_ref, f1_ref, f2_ref, f3_ref, f4_ref, w_ref,
               r_ref, s_ref, m_ref):
    attr = attr_ref[...]
    t1 = jnp.maximum(attr @ f1_ref[...], 0.0)
    t2 = jnp.tanh(attr @ f2_ref[...]) * jnp.tanh(attr @ f3_ref[...])
    tmp = jnp.concatenate([t1, t2], axis=1)
    ea = jnp.maximum(tmp @ f4_ref[...], 0.0)
    z = jnp.dot(xj_ref[...], w_ref[...], preferred_element_type=jnp.float32)
    ea_rep = jnp.dot(ea, r_ref[...], preferred_element_type=jnp.float32)
    acc = jnp.dot(z * ea_rep, s_ref[...], preferred_element_type=jnp.float32)
    mpad = jnp.zeros((acc.shape[0], 128 - _NOUT1), jnp.float32)
    m_ref[...] = jnp.concatenate([acc, mpad], axis=1)


@functools.partial(jax.jit, static_argnames=("ninp",))
def _edge_messages(edge_attr, xj, f1, f2, f3, f4, w_all, rep_mat, sum_mat, ninp):
    grid = _E // _TE
    return pl.pallas_call(
        _edge_body,
        grid=(grid,),
        in_specs=[
            pl.BlockSpec((_TE, _NE), lambda i: (i, 0)),
            pl.BlockSpec((_TE, ninp), lambda i: (i, 0)),
            pl.BlockSpec((_NE, 2 * _NE), lambda i: (0, 0)),
            pl.BlockSpec((_NE, 2 * _NE), lambda i: (0, 0)),
            pl.BlockSpec((_NE, 2 * _NE), lambda i: (0, 0)),
            pl.BlockSpec((4 * _NE, _NE), lambda i: (0, 0)),
            pl.BlockSpec((ninp, _NE * _NOUT1), lambda i: (0, 0)),
            pl.BlockSpec((_NE, _NE * _NOUT1), lambda i: (0, 0)),
            pl.BlockSpec((_NE * _NOUT1, _NOUT1), lambda i: (0, 0)),
        ],
        out_specs=pl.BlockSpec((_TE, 128), lambda i: (i, 0)),
        out_shape=jax.ShapeDtypeStruct((_E, 128), jnp.float32),
    )(edge_attr, xj, f1, f2, f3, f4, w_all, rep_mat, sum_mat)


_TN = 2000


def _node_body(agg2_ref, x_ref, cb_ref, f11_ref, b11_ref, f12_ref, b12_ref, h_ref):
    agg = agg2_ref[:, 0:_NOUT1]
    out1 = jnp.maximum(agg + cb_ref[...], 0.0)
    xt = x_ref[...]
    a = jnp.tanh(jnp.dot(xt, f11_ref[...], preferred_element_type=jnp.float32) + b11_ref[...])
    b = jnp.tanh(jnp.dot(xt, f12_ref[...], preferred_element_type=jnp.float32) + b12_ref[...])
    pad = jnp.zeros((out1.shape[0], 128 - _NOUT1 - _NOUT2), jnp.float32)
    h_ref[...] = jnp.concatenate([out1, a * b, pad], axis=1)


@functools.partial(jax.jit, static_argnames=("ninp",))
def _node_update(agg2, x, cb, f11, b11, f12, b12, ninp):
    grid = _N // _TN
    return pl.pallas_call(
        _node_body,
        grid=(grid,),
        in_specs=[
            pl.BlockSpec((_TN, 128), lambda i: (i, 0)),
            pl.BlockSpec((_TN, ninp), lambda i: (i, 0)),
            pl.BlockSpec((1, _NOUT1), lambda i: (0, 0)),
            pl.BlockSpec((ninp, _NOUT2), lambda i: (0, 0)),
            pl.BlockSpec((1, _NOUT2), lambda i: (0, 0)),
            pl.BlockSpec((ninp, _NOUT2), lambda i: (0, 0)),
            pl.BlockSpec((1, _NOUT2), lambda i: (0, 0)),
        ],
        out_specs=pl.BlockSpec((_TN, 128), lambda i: (i, 0)),
        out_shape=jax.ShapeDtypeStruct((_N, 128), jnp.float32),
    )(agg2, x, cb, f11, b11, f12, b12)


def _head_body(h_ref, batch_ref, w1_ref, b1_ref, w2_ref, b2_ref, out_ref,
               sums_ref, cnts_ref):
    i = pl.program_id(0)

    @pl.when(i == 0)
    def _():
        sums_ref[...] = jnp.zeros_like(sums_ref)
        cnts_ref[...] = jnp.zeros_like(cnts_ref)

    onehot_t = (lax.broadcasted_iota(jnp.int32, (_NG, _TN), 0)
                == batch_ref[0]).astype(jnp.float32)
    sums_ref[...] += jnp.dot(onehot_t, h_ref[...], preferred_element_type=jnp.float32)
    cnts_ref[...] += jnp.sum(onehot_t, axis=1, keepdims=True)

    @pl.when(i == pl.num_programs(0) - 1)
    def _():
        g = sums_ref[...] / jnp.maximum(cnts_ref[...], 1.0)
        a = jnp.maximum(jnp.dot(g, w1_ref[...], preferred_element_type=jnp.float32)
                        + b1_ref[...], 0.0)
        out_ref[...] = jnp.dot(a, w2_ref[...], preferred_element_type=jnp.float32) + b2_ref[...]


@jax.jit
def _head(h, batch_row, w1, b1, w2, b2):
    grid = _N // _TN
    nin = 128
    return pl.pallas_call(
        _head_body,
        grid=(grid,),
        in_specs=[
            pl.BlockSpec((_TN, nin), lambda i: (i, 0)),
            pl.BlockSpec((1, 1, _TN), lambda i: (i, 0, 0)),
            pl.BlockSpec((nin, 10), lambda i: (0, 0)),
            pl.BlockSpec((1, 10), lambda i: (0, 0)),
            pl.BlockSpec((10, 1), lambda i: (0, 0)),
            pl.BlockSpec((1, 1), lambda i: (0, 0)),
        ],
        out_specs=pl.BlockSpec((_NG, 1), lambda i: (0, 0)),
        out_shape=jax.ShapeDtypeStruct((_NG, 1), jnp.float32),
        scratch_shapes=[
            pltpu.VMEM((_NG, 128), jnp.float32),
            pltpu.VMEM((_NG, 1), jnp.float32),
        ],
    )(h, batch_row, w1, b1, w2, b2)


def kernel(x, edge_index, edge_attr, batch, params):
    src = edge_index[0]
    dst = edge_index[1]
    zeros_n = jnp.zeros((_NP, 128), jnp.float32)
    batch_row = batch.reshape(_N // _TN, 1, _TN)

    ii = jnp.arange(_NE * _NOUT1, dtype=jnp.int32)
    rep_mat = (ii[None, :] // _NOUT1 == jnp.arange(_NE, dtype=jnp.int32)[:, None]
               ).astype(jnp.float32)
    sum_mat = (ii[:, None] % _NOUT1 == jnp.arange(_NOUT1, dtype=jnp.int32)[None, :]
               ).astype(jnp.float32)

    def pad_rows(w):
        return jnp.pad(w, ((0, 128 - w.shape[0]), (0, 0)))

    h = x
    for lp in (params['l1'], params['l2'], params['l3']):
        ninp = lp['conv_w'].shape[1]
        w_all = pad_rows(jnp.transpose(lp['conv_w'], (1, 0, 2)).reshape(ninp, _NE * _NOUT1))
        xj = _sc_gather(h, src, 128)
        m = _edge_messages(edge_attr, xj, lp['fc1_1'], lp['fc1_2'], lp['fc1_3'],
                           lp['fc1_4'], w_all, rep_mat, sum_mat, 128)
        agg2 = _scatter_dispatch(m, dst, zeros_n)
        h = _node_update(agg2, h, lp['conv_b'].reshape(1, _NOUT1),
                         pad_rows(lp['fc11_w']), lp['fc11_b'].reshape(1, _NOUT2),
                         pad_rows(lp['fc12_w']), lp['fc12_b'].reshape(1, _NOUT2), 128)

    return _head(h, batch_row, pad_rows(params['fc1_w']), params['fc1_b'].reshape(1, 10),
                 params['fc2_w'], params['fc2_b'].reshape(1, 1))
